# baseline (speedup 1.0000x reference)
import jax
import jax.numpy as jnp
from jax import lax
from jax.experimental import pallas as pl
from jax.experimental.pallas import tpu as pltpu

B = 4096
D = 64
NSEM = 8
UNROLL = 8


def _body(idx_s, table_hbm, out_ref, rows_v, *sems):
    def issue(jb, _):
        for u in range(UNROLL):
            j = jb * UNROLL + u
            pltpu.make_async_copy(
                table_hbm.at[pl.ds(idx_s[j], 1), :],
                rows_v.at[pl.ds(j, 1), :], sems[u % NSEM]).start()
        return 0

    lax.fori_loop(0, B // UNROLL, issue, 0)
    for q in range(NSEM):
        pltpu.make_async_copy(
            table_hbm.at[pl.ds(0, B // NSEM), :],
            rows_v.at[pl.ds(q * (B // NSEM), B // NSEM), :],
            sems[q]).wait()

    x = rows_v[...]
    rinv = lax.rsqrt(jnp.sum(x * x, axis=1, keepdims=True))
    out_ref[...] = (x * rinv).T


def kernel(nodes, table):
    grid_spec = pltpu.PrefetchScalarGridSpec(
        num_scalar_prefetch=1,
        grid=(1,),
        in_specs=[pl.BlockSpec(memory_space=pl.ANY)],
        out_specs=pl.BlockSpec((D, B), lambda i, idx: (0, 0)),
        scratch_shapes=[
            pltpu.VMEM((B, D), jnp.float32),
        ] + [pltpu.SemaphoreType.DMA] * NSEM,
    )
    return pl.pallas_call(
        _body,
        grid_spec=grid_spec,
        out_shape=jax.ShapeDtypeStruct((D, B), jnp.float32),
    )(nodes.astype(jnp.int32), table)
